# baseline (speedup 1.0000x reference)
import functools

import jax
import jax.numpy as jnp
from jax import lax
from jax.experimental import pallas as pl
from jax.experimental.pallas import tpu as pltpu
from jax.experimental.pallas import tpu_sc as plsc

N_NODES = 10000
N_EDGES = 320000
D = 128
L = 16
NC, NS = 2, 16
NW = NC * NS
EPW = N_EDGES // NW
C = 128
NCHUNK = -(-EPW // C)
NPAIR = (NCHUNK + 1) // 2


def _sc_body(z_hbm, row_hbm, col_hbm, out_hbm,
             idx_r, idx_c, rows, cols, out_v, tr,
             sem_r0, sem_c0, sem_r1, sem_c1):
    cid = lax.axis_index("c")
    sid = lax.axis_index("s")
    wid = sid * NC + cid
    ebase = pl.multiple_of(wid * EPW, 8)

    lane = lax.broadcasted_iota(jnp.int32, (L,), 0)
    sems = [(sem_r0, sem_c0), (sem_r1, sem_c1)]
    last_off = EPW - C

    pltpu.sync_copy(row_hbm.at[pl.ds(ebase, EPW)], idx_r)
    pltpu.sync_copy(col_hbm.at[pl.ds(ebase, EPW)], idx_c)

    def chunk_off(k):
        return pl.multiple_of(jnp.minimum(k * C, last_off), 8)

    def start_gather(b, k):
        off = chunk_off(k)
        sr, sc_ = sems[b]
        pltpu.make_async_copy(
            z_hbm.at[idx_r.at[pl.ds(off, C)]], rows.at[b], sr).start()
        pltpu.make_async_copy(
            z_hbm.at[idx_c.at[pl.ds(off, C)]], cols.at[b], sc_).start()

    def wait_gather(b):
        sr, sc_ = sems[b]
        pltpu.make_async_copy(z_hbm.at[pl.ds(0, C)], rows.at[b], sr).wait()
        pltpu.make_async_copy(z_hbm.at[pl.ds(0, C)], cols.at[b], sc_).wait()

    tr_stride = L + 1
    tr_base = lane * tr_stride

    trs = L * (L + 1) + 8

    def compute(b, k):
        obase = chunk_off(k)

        def store_group(i, sbase):
            for e2 in range(L):
                e = i * L + e2
                acc = rows[b, e, pl.ds(0, L)] * cols[b, e, pl.ds(0, L)]
                for j in range(1, D // L):
                    acc = acc + (rows[b, e, pl.ds(j * L, L)]
                                 * cols[b, e, pl.ds(j * L, L)])
                tr[pl.ds(sbase + e2 * tr_stride, L)] = acc

        def reduce_group(i, sbase):
            p = [plsc.load_gather(tr, [sbase + tr_base + k2])
                 for k2 in range(L)]
            while len(p) > 1:
                p = [p[k2] + p[k2 + 1] for k2 in range(0, len(p), 2)]
            out_v[pl.ds(obase + i * L, L)] = 1.0 / (1.0 + jnp.exp(-p[0]))

        store_group(jnp.int32(0), jnp.int32(0))

        def body16(i, carry):
            store_group(i, (i & 1) * trs)
            reduce_group(i - 1, ((i - 1) & 1) * trs)
            return carry

        lax.fori_loop(1, C // L, body16, 0, unroll=False)
        last = jnp.int32(C // L - 1)
        reduce_group(last, (last & 1) * trs)

    start_gather(0, jnp.int32(0))

    def pair(p, carry):
        k0 = 2 * p
        start_gather(1, k0 + 1)
        wait_gather(0)
        compute(0, k0)
        start_gather(0, k0 + 2)
        wait_gather(1)
        compute(1, k0 + 1)
        return carry

    lax.fori_loop(0, NPAIR, pair, 0, unroll=False)
    wait_gather(0)

    pltpu.sync_copy(out_v, out_hbm.at[pl.ds(ebase, EPW)])


@functools.partial(
    pl.kernel,
    out_type=jax.ShapeDtypeStruct((N_EDGES,), jnp.float32),
    mesh=plsc.VectorSubcoreMesh(core_axis_name="c", subcore_axis_name="s"),
    compiler_params=pltpu.CompilerParams(needs_layout_passes=False),
    scratch_types=[
        pltpu.VMEM((EPW,), jnp.int32),
        pltpu.VMEM((EPW,), jnp.int32),
        pltpu.VMEM((2, C, D), jnp.float32),
        pltpu.VMEM((2, C, D), jnp.float32),
        pltpu.VMEM((EPW,), jnp.float32),
        pltpu.VMEM((2 * (L * (L + 1) + 8),), jnp.float32),
        pltpu.SemaphoreType.DMA,
        pltpu.SemaphoreType.DMA,
        pltpu.SemaphoreType.DMA,
        pltpu.SemaphoreType.DMA,
    ],
)
def _edge_probs_sc(z_hbm, row_hbm, col_hbm, out_hbm, *scratch):
    _sc_body(z_hbm, row_hbm, col_hbm, out_hbm, *scratch)


def kernel(z, edge_index):
    row = edge_index[0].astype(jnp.int32)
    col = edge_index[1].astype(jnp.int32)
    return _edge_probs_sc(z, row, col)


# speedup vs baseline: 1.1251x; 1.1251x over previous
import functools

import jax
import jax.numpy as jnp
from jax import lax
from jax.experimental import pallas as pl
from jax.experimental.pallas import tpu as pltpu
from jax.experimental.pallas import tpu_sc as plsc

N_NODES = 10000
N_EDGES = 320000
D = 128
L = 16
W = D // 2
NC, NS = 2, 16
NW = NC * NS
EPW = N_EDGES // NW
C = 128
NCHUNK = -(-EPW // C)
NPAIR = (NCHUNK + 1) // 2


def _sc_body(z_hbm, row_hbm, col_hbm, out_hbm,
             idx_r, idx_c, rows, cols, out_v, tr,
             sem_r0, sem_c0, sem_r1, sem_c1):
    cid = lax.axis_index("c")
    sid = lax.axis_index("s")
    wid = sid * NC + cid
    ebase = pl.multiple_of(wid * EPW, 8)

    lane = lax.broadcasted_iota(jnp.int32, (L,), 0)
    sems = [(sem_r0, sem_c0), (sem_r1, sem_c1)]
    last_off = EPW - C

    pltpu.sync_copy(row_hbm.at[pl.ds(ebase, EPW)], idx_r)
    pltpu.sync_copy(col_hbm.at[pl.ds(ebase, EPW)], idx_c)

    def chunk_off(k):
        return pl.multiple_of(jnp.minimum(k * C, last_off), 8)

    def start_gather(b, k):
        off = chunk_off(k)
        sr, sc_ = sems[b]
        pltpu.make_async_copy(
            z_hbm.at[idx_r.at[pl.ds(off, C)]], rows.at[b], sr).start()
        pltpu.make_async_copy(
            z_hbm.at[idx_c.at[pl.ds(off, C)]], cols.at[b], sc_).start()

    def wait_gather(b):
        sr, sc_ = sems[b]
        pltpu.make_async_copy(z_hbm.at[pl.ds(0, C)], rows.at[b], sr).wait()
        pltpu.make_async_copy(z_hbm.at[pl.ds(0, C)], cols.at[b], sc_).wait()

    tr_stride = L + 1
    tr_base = lane * tr_stride
    himask = jnp.int32(-65536)

    def edge_dot(b, e):
        parts = []
        for j in range(W // L):
            qr = plsc.bitcast(rows[b, e, pl.ds(j * L, L)], jnp.bfloat16)
            qc = plsc.bitcast(cols[b, e, pl.ds(j * L, L)], jnp.bfloat16)
            pw = plsc.bitcast(qr * qc, jnp.int32)
            lo = plsc.bitcast(lax.shift_left(pw, jnp.int32(16)), jnp.float32)
            hi = plsc.bitcast(pw & himask, jnp.float32)
            parts.append(lo + hi)
        while len(p := parts) > 1:
            parts = [p[k] + p[k + 1] for k in range(0, len(p), 2)]
        return parts[0]

    def compute(b, k):
        obase = chunk_off(k)

        def body16(i, carry):
            for e2 in range(L):
                tr[pl.ds(e2 * tr_stride, L)] = edge_dot(b, i * L + e2)
            p = [plsc.load_gather(tr, [tr_base + k2]) for k2 in range(L)]
            while len(p) > 1:
                p = [p[k2] + p[k2 + 1] for k2 in range(0, len(p), 2)]
            out_v[pl.ds(obase + i * L, L)] = 1.0 / (1.0 + jnp.exp(-p[0]))
            return carry

        lax.fori_loop(0, C // L, body16, 0, unroll=False)

    start_gather(0, jnp.int32(0))

    def pair(p, carry):
        k0 = 2 * p
        start_gather(1, k0 + 1)
        wait_gather(0)
        compute(0, k0)
        start_gather(0, k0 + 2)
        wait_gather(1)
        compute(1, k0 + 1)
        return carry

    lax.fori_loop(0, NPAIR, pair, 0, unroll=False)
    wait_gather(0)

    pltpu.sync_copy(out_v, out_hbm.at[pl.ds(ebase, EPW)])


@functools.partial(
    pl.kernel,
    out_type=jax.ShapeDtypeStruct((N_EDGES,), jnp.float32),
    mesh=plsc.VectorSubcoreMesh(core_axis_name="c", subcore_axis_name="s"),
    compiler_params=pltpu.CompilerParams(
        needs_layout_passes=False, use_tc_tiling_on_sc=False),
    scratch_types=[
        pltpu.VMEM((EPW,), jnp.int32),
        pltpu.VMEM((EPW,), jnp.int32),
        pltpu.VMEM((2, C, W), jnp.int32),
        pltpu.VMEM((2, C, W), jnp.int32),
        pltpu.VMEM((EPW,), jnp.float32),
        pltpu.VMEM((L * (L + 1) + 8,), jnp.float32),
        pltpu.SemaphoreType.DMA,
        pltpu.SemaphoreType.DMA,
        pltpu.SemaphoreType.DMA,
        pltpu.SemaphoreType.DMA,
    ],
)
def _edge_probs_sc(z_hbm, row_hbm, col_hbm, out_hbm, *scratch):
    _sc_body(z_hbm, row_hbm, col_hbm, out_hbm, *scratch)


def kernel(z, edge_index):
    row = edge_index[0].astype(jnp.int32)
    col = edge_index[1].astype(jnp.int32)
    z_pk = lax.bitcast_convert_type(
        z.astype(jnp.bfloat16).reshape(N_NODES, W, 2), jnp.int32)
    return _edge_probs_sc(z_pk, row, col)


# speedup vs baseline: 1.2184x; 1.0829x over previous
import functools

import jax
import jax.numpy as jnp
from jax import lax
from jax.experimental import pallas as pl
from jax.experimental.pallas import tpu as pltpu
from jax.experimental.pallas import tpu_sc as plsc

N_NODES = 10000
N_EDGES = 320000
D = 128
L = 16
NC, NS = 2, 16
NW = NC * NS
EPW = N_EDGES // NW
C = 128
NCHUNK = -(-EPW // C)
NTRI = -(-NCHUNK // 3)


def _sc_body(z_hbm, row_hbm, col_hbm, out_hbm,
             idx_r, idx_c, rows, cols, out_v, tr,
             sem_r0, sem_c0, sem_r1, sem_c1, sem_r2, sem_c2):
    cid = lax.axis_index("c")
    sid = lax.axis_index("s")
    wid = sid * NC + cid
    ebase = pl.multiple_of(wid * EPW, 8)

    lane = lax.broadcasted_iota(jnp.int32, (L,), 0)
    sems = [(sem_r0, sem_c0), (sem_r1, sem_c1), (sem_r2, sem_c2)]
    last_off = EPW - C

    pltpu.sync_copy(row_hbm.at[pl.ds(ebase, EPW)], idx_r)
    pltpu.sync_copy(col_hbm.at[pl.ds(ebase, EPW)], idx_c)

    def chunk_off(k):
        return pl.multiple_of(jnp.minimum(k * C, last_off), 8)

    def start_gather(b, k):
        off = chunk_off(k)
        sr, sc_ = sems[b]
        pltpu.make_async_copy(
            z_hbm.at[idx_r.at[pl.ds(off, C)]], rows.at[b], sr).start()
        pltpu.make_async_copy(
            z_hbm.at[idx_c.at[pl.ds(off, C)]], cols.at[b], sc_).start()

    def wait_gather(b):
        sr, sc_ = sems[b]
        pltpu.make_async_copy(z_hbm.at[pl.ds(0, C)], rows.at[b], sr).wait()
        pltpu.make_async_copy(z_hbm.at[pl.ds(0, C)], cols.at[b], sc_).wait()

    tr_stride = L + 1
    tr_base = lane * tr_stride

    def compute(b, k):
        obase = chunk_off(k)

        def body16(i, carry):
            for e2 in range(L):
                e = i * L + e2
                acc = rows[b, e, pl.ds(0, L)] * cols[b, e, pl.ds(0, L)]
                for j in range(1, D // L):
                    acc = acc + (rows[b, e, pl.ds(j * L, L)]
                                 * cols[b, e, pl.ds(j * L, L)])
                tr[pl.ds(e2 * tr_stride, L)] = acc
            p = [plsc.load_gather(tr, [tr_base + k2]) for k2 in range(L)]
            while len(p) > 1:
                p = [p[k2] + p[k2 + 1] for k2 in range(0, len(p), 2)]
            out_v[pl.ds(obase + i * L, L)] = 1.0 / (1.0 + jnp.exp(-p[0]))
            return carry

        lax.fori_loop(0, C // L, body16, 0, unroll=False)

    start_gather(0, jnp.int32(0))
    start_gather(1, jnp.int32(1))

    def tri(t, carry):
        k0 = 3 * t
        start_gather(2, k0 + 2)
        wait_gather(0)
        compute(0, k0)
        start_gather(0, k0 + 3)
        wait_gather(1)
        compute(1, k0 + 1)
        start_gather(1, k0 + 4)
        wait_gather(2)
        compute(2, k0 + 2)
        return carry

    lax.fori_loop(0, NTRI, tri, 0, unroll=False)
    wait_gather(0)
    wait_gather(1)

    pltpu.sync_copy(out_v, out_hbm.at[pl.ds(ebase, EPW)])


@functools.partial(
    pl.kernel,
    out_type=jax.ShapeDtypeStruct((N_EDGES,), jnp.float32),
    mesh=plsc.VectorSubcoreMesh(core_axis_name="c", subcore_axis_name="s"),
    compiler_params=pltpu.CompilerParams(needs_layout_passes=False),
    scratch_types=[
        pltpu.VMEM((EPW,), jnp.int32),
        pltpu.VMEM((EPW,), jnp.int32),
        pltpu.VMEM((3, C, D), jnp.float32),
        pltpu.VMEM((3, C, D), jnp.float32),
        pltpu.VMEM((EPW,), jnp.float32),
        pltpu.VMEM((L * (L + 1) + 8,), jnp.float32),
        pltpu.SemaphoreType.DMA,
        pltpu.SemaphoreType.DMA,
        pltpu.SemaphoreType.DMA,
        pltpu.SemaphoreType.DMA,
        pltpu.SemaphoreType.DMA,
        pltpu.SemaphoreType.DMA,
    ],
)
def _edge_probs_sc(z_hbm, row_hbm, col_hbm, out_hbm, *scratch):
    _sc_body(z_hbm, row_hbm, col_hbm, out_hbm, *scratch)


def kernel(z, edge_index):
    row = edge_index[0].astype(jnp.int32)
    col = edge_index[1].astype(jnp.int32)
    return _edge_probs_sc(z, row, col)
